# Initial kernel scaffold; baseline (speedup 1.0000x reference)
import jax
import jax.numpy as jnp
from jax.experimental import pallas as pl


def kernel(input, h_prev, m_prev, W_h, b_h, ln_g, ln_b, W_a, b_a, W_b, b_b, W_va, b_va, W_vb, b_vb):
    raise NotImplementedError("write your pallas kernel here")


# speedup vs baseline: 3.7410x; 3.7410x over previous
import jax
import jax.numpy as jnp
from jax.experimental import pallas as pl
from jax.experimental.pallas import tpu as pltpu

B = 16384
INPUT, HID, MEM, K = 512, 1024, 1152, 2
SQ = 48
CH = SQ // K
LN_EPS = 1e-5
P_EPS = 1e-12

BLOCK = 512
NP = 384


def _nru_kernel(x_ref, hp_ref, mp_ref,
                whi_ref, whh_ref, whm_ref, bh_ref, g_ref, bln_ref,
                w2h_ref, w2m_ref, b2_ref, rr_ref, tt_ref,
                ho_ref, mo_ref):
    bf16 = jnp.bfloat16
    f32 = jnp.float32
    xb = x_ref[...].astype(bf16)
    hpb = hp_ref[...].astype(bf16)
    mp = mp_ref[...]
    mpb = mp.astype(bf16)

    z = (jnp.dot(xb, whi_ref[...], preferred_element_type=f32)
         + jnp.dot(hpb, whh_ref[...], preferred_element_type=f32)
         + jnp.dot(mpb, whm_ref[...], preferred_element_type=f32)
         + bh_ref[...])

    mu = jnp.mean(z, axis=-1, keepdims=True)
    ez2 = jnp.mean(z * z, axis=-1, keepdims=True)
    var = ez2 - mu * mu
    h = jnp.maximum((z - mu) * jax.lax.rsqrt(var + LN_EPS) * g_ref[...] + bln_ref[...], 0.0)
    ho_ref[...] = h

    t = (jnp.dot(h.astype(bf16), w2h_ref[...], preferred_element_type=f32)
         + jnp.dot(mpb, w2m_ref[...], preferred_element_type=f32)
         + b2_ref[...])
    t0 = t[:, 0:128]
    t1 = t[:, 128:256]
    t2 = t[:, 256:384]

    def p5(v):
        a = jnp.abs(v)
        a2 = a * a
        return a2 * a2 * a

    lane = jax.lax.broadcasted_iota(jnp.int32, (1, 128), 1)
    a5 = p5(t0)
    c5 = p5(t1)
    zero = jnp.zeros_like(a5)
    s10a = jnp.sum(jnp.where(lane < CH, a5, zero), axis=1, keepdims=True)
    s11a = jnp.sum(jnp.where((lane >= CH) & (lane < SQ), a5, zero), axis=1, keepdims=True)
    s10b = jnp.sum(jnp.where((lane >= SQ) & (lane < SQ + CH), a5, zero), axis=1, keepdims=True)
    s11b = jnp.sum(jnp.where((lane >= SQ + CH) & (lane < 2 * SQ), a5, zero), axis=1, keepdims=True)
    s2a = jnp.sum(jnp.where(lane < SQ, c5, zero), axis=1, keepdims=True)
    s2b = jnp.sum(jnp.where((lane >= SQ) & (lane < 2 * SQ), c5, zero), axis=1, keepdims=True)

    pp = jnp.where(lane < CH, s10a * s2a,
                   jnp.where(lane < SQ, s11a * s2a,
                             jnp.where(lane < SQ + CH, s10b * s2b, s11b * s2b)))
    nn = jnp.maximum(jnp.exp2(jnp.log2(pp) * 0.2), P_EPS)
    w_all = t2 * t0 * (1.0 / K) / nn

    wa = w_all[:, 0:SQ].astype(bf16)
    wb = w_all[:, SQ:2 * SQ].astype(bf16)
    u2a = t1[:, 0:SQ].astype(bf16)
    u2b = t1[:, SQ:2 * SQ].astype(bf16)

    rr = rr_ref[...]
    tt = tt_ref[...]
    add = (jnp.dot(wa, rr, preferred_element_type=f32)
           * jnp.dot(u2a, tt, preferred_element_type=f32))
    fgt = (jnp.dot(wb, rr, preferred_element_type=f32)
           * jnp.dot(u2b, tt, preferred_element_type=f32))
    mo_ref[...] = mp + add - fgt


@jax.jit
def kernel(input, h_prev, m_prev, W_h, b_h, ln_g, ln_b, W_a, b_a, W_b, b_b, W_va, b_va, W_vb, b_vb):
    f32 = jnp.float32
    bf16 = jnp.bfloat16
    wt = W_h.T.astype(bf16)
    whi = wt[:INPUT]
    whh = wt[INPUT:INPUT + HID]
    whm = wt[INPUT + HID:]

    zc32 = jnp.zeros((32, HID + MEM), f32)
    rows = jnp.concatenate([
        W_va[0:SQ], W_vb[0:SQ], zc32,
        W_va[SQ:2 * SQ], W_vb[SQ:2 * SQ], zc32,
        jnp.tile(W_a[0:1], (CH, 1)), jnp.tile(W_a[1:2], (CH, 1)),
        jnp.tile(W_b[0:1], (CH, 1)), jnp.tile(W_b[1:2], (CH, 1)), zc32,
    ], axis=0)
    w2 = rows.T.astype(bf16)
    w2h = w2[:HID]
    w2m = w2[HID:]
    zb32 = jnp.zeros((32,), f32)
    b2 = jnp.concatenate([
        b_va[0:SQ], b_vb[0:SQ], zb32,
        b_va[SQ:2 * SQ], b_vb[SQ:2 * SQ], zb32,
        jnp.tile(b_a[0:1], (CH,)), jnp.tile(b_a[1:2], (CH,)),
        jnp.tile(b_b[0:1], (CH,)), jnp.tile(b_b[1:2], (CH,)), zb32,
    ]).reshape(1, NP)

    m_ids = jax.lax.iota(jnp.int32, MEM)
    r_ids = jax.lax.iota(jnp.int32, SQ)
    rrmat = (m_ids[None, :] // SQ == r_ids[:, None] % CH).astype(bf16)
    ttmat = (m_ids[None, :] % SQ == r_ids[:, None]).astype(bf16)

    n_cores = 2
    steps = B // BLOCK // n_cores
    grid = (n_cores, steps)
    row = lambda c, j: (c * steps + j, 0)
    const = lambda c, j: (0, 0)

    h, m = pl.pallas_call(
        _nru_kernel,
        grid=grid,
        in_specs=[
            pl.BlockSpec((BLOCK, INPUT), row),
            pl.BlockSpec((BLOCK, HID), row),
            pl.BlockSpec((BLOCK, MEM), row),
            pl.BlockSpec((INPUT, HID), const),
            pl.BlockSpec((HID, HID), const),
            pl.BlockSpec((MEM, HID), const),
            pl.BlockSpec((1, HID), const),
            pl.BlockSpec((1, HID), const),
            pl.BlockSpec((1, HID), const),
            pl.BlockSpec((HID, NP), const),
            pl.BlockSpec((MEM, NP), const),
            pl.BlockSpec((1, NP), const),
            pl.BlockSpec((SQ, MEM), const),
            pl.BlockSpec((SQ, MEM), const),
        ],
        out_specs=[
            pl.BlockSpec((BLOCK, HID), row),
            pl.BlockSpec((BLOCK, MEM), row),
        ],
        out_shape=[
            jax.ShapeDtypeStruct((B, HID), f32),
            jax.ShapeDtypeStruct((B, MEM), f32),
        ],
        compiler_params=pltpu.CompilerParams(
            dimension_semantics=(pltpu.CORE_PARALLEL, "arbitrary"),
            vmem_limit_bytes=56 * 1024 * 1024,
        ),
    )(input, h_prev, m_prev,
      whi, whh, whm, b_h.reshape(1, HID), ln_g.reshape(1, HID), ln_b.reshape(1, HID),
      w2h, w2m, b2, rrmat, ttmat)
    return h, m


# speedup vs baseline: 3.9922x; 1.0672x over previous
import jax
import jax.numpy as jnp
from jax.experimental import pallas as pl
from jax.experimental.pallas import tpu as pltpu

B = 16384
INPUT, HID, MEM, K = 512, 1024, 1152, 2
SQ = 48
CH = SQ // K
LN_EPS = 1e-5
P_EPS = 1e-12

BLOCK = 1024
NP = 384

_DN = (((1,), (1,)), ((), ()))


def _nru_kernel(x_ref, hp_ref, mp_ref,
                wh_ref, bh_ref, g_ref, bln_ref,
                w2_ref, b2_ref, rr_ref, tt_ref,
                ho_ref, mo_ref):
    bf16 = jnp.bfloat16
    f32 = jnp.float32
    xb = x_ref[...].astype(bf16)
    hpb = hp_ref[...].astype(bf16)
    mp = mp_ref[...]
    mpb = mp.astype(bf16)

    wh = wh_ref[...]
    z = (jax.lax.dot_general(xb, wh[:, 0:INPUT], _DN, preferred_element_type=f32)
         + jax.lax.dot_general(hpb, wh[:, INPUT:INPUT + HID], _DN, preferred_element_type=f32)
         + jax.lax.dot_general(mpb, wh[:, INPUT + HID:], _DN, preferred_element_type=f32)
         + bh_ref[...])

    mu = jnp.mean(z, axis=-1, keepdims=True)
    ez2 = jnp.mean(z * z, axis=-1, keepdims=True)
    var = ez2 - mu * mu
    h = jnp.maximum((z - mu) * jax.lax.rsqrt(var + LN_EPS) * g_ref[...] + bln_ref[...], 0.0)
    ho_ref[...] = h

    w2 = w2_ref[...]
    t = (jax.lax.dot_general(h.astype(bf16), w2[:, 0:HID], _DN, preferred_element_type=f32)
         + jax.lax.dot_general(mpb, w2[:, HID:], _DN, preferred_element_type=f32)
         + b2_ref[...])
    t0 = t[:, 0:128]
    t1 = t[:, 128:256]
    t2 = t[:, 256:384]

    def p5(v):
        a = jnp.abs(v)
        a2 = a * a
        return a2 * a2 * a

    lane = jax.lax.broadcasted_iota(jnp.int32, (1, 128), 1)
    a5 = p5(t0)
    c5 = p5(t1)
    zero = jnp.zeros_like(a5)
    s10a = jnp.sum(jnp.where(lane < CH, a5, zero), axis=1, keepdims=True)
    s11a = jnp.sum(jnp.where((lane >= CH) & (lane < SQ), a5, zero), axis=1, keepdims=True)
    s10b = jnp.sum(jnp.where((lane >= SQ) & (lane < SQ + CH), a5, zero), axis=1, keepdims=True)
    s11b = jnp.sum(jnp.where((lane >= SQ + CH) & (lane < 2 * SQ), a5, zero), axis=1, keepdims=True)
    s2a = jnp.sum(jnp.where(lane < SQ, c5, zero), axis=1, keepdims=True)
    s2b = jnp.sum(jnp.where((lane >= SQ) & (lane < 2 * SQ), c5, zero), axis=1, keepdims=True)

    pp = jnp.where(lane < CH, s10a * s2a,
                   jnp.where(lane < SQ, s11a * s2a,
                             jnp.where(lane < SQ + CH, s10b * s2b, s11b * s2b)))
    nn = jnp.maximum(jnp.exp2(jnp.log2(pp) * 0.2), P_EPS)
    w_all = t2 * t0 * (1.0 / K) / nn

    wa = w_all[:, 0:SQ].astype(bf16)
    wb = w_all[:, SQ:2 * SQ].astype(bf16)
    u2a = t1[:, 0:SQ].astype(bf16)
    u2b = t1[:, SQ:2 * SQ].astype(bf16)

    rr = rr_ref[...]
    tt = tt_ref[...]
    add = (jnp.dot(wa, rr, preferred_element_type=f32)
           * jnp.dot(u2a, tt, preferred_element_type=f32))
    fgt = (jnp.dot(wb, rr, preferred_element_type=f32)
           * jnp.dot(u2b, tt, preferred_element_type=f32))
    mo_ref[...] = mp + add - fgt


@jax.jit
def kernel(input, h_prev, m_prev, W_h, b_h, ln_g, ln_b, W_a, b_a, W_b, b_b, W_va, b_va, W_vb, b_vb):
    f32 = jnp.float32
    bf16 = jnp.bfloat16
    wh = W_h.astype(bf16)

    zc32 = jnp.zeros((32, HID + MEM), f32)
    rows = jnp.concatenate([
        W_va[0:SQ], W_vb[0:SQ], zc32,
        W_va[SQ:2 * SQ], W_vb[SQ:2 * SQ], zc32,
        jnp.tile(W_a[0:1], (CH, 1)), jnp.tile(W_a[1:2], (CH, 1)),
        jnp.tile(W_b[0:1], (CH, 1)), jnp.tile(W_b[1:2], (CH, 1)), zc32,
    ], axis=0).astype(bf16)
    zb32 = jnp.zeros((32,), f32)
    b2 = jnp.concatenate([
        b_va[0:SQ], b_vb[0:SQ], zb32,
        b_va[SQ:2 * SQ], b_vb[SQ:2 * SQ], zb32,
        jnp.tile(b_a[0:1], (CH,)), jnp.tile(b_a[1:2], (CH,)),
        jnp.tile(b_b[0:1], (CH,)), jnp.tile(b_b[1:2], (CH,)), zb32,
    ]).reshape(1, NP)

    m_ids = jax.lax.iota(jnp.int32, MEM)
    r_ids = jax.lax.iota(jnp.int32, SQ)
    rrmat = (m_ids[None, :] // SQ == r_ids[:, None] % CH).astype(bf16)
    ttmat = (m_ids[None, :] % SQ == r_ids[:, None]).astype(bf16)

    grid = (B // BLOCK,)
    row = lambda i: (i, 0)
    const = lambda i: (0, 0)

    h, m = pl.pallas_call(
        _nru_kernel,
        grid=grid,
        in_specs=[
            pl.BlockSpec((BLOCK, INPUT), row),
            pl.BlockSpec((BLOCK, HID), row),
            pl.BlockSpec((BLOCK, MEM), row),
            pl.BlockSpec((HID, INPUT + HID + MEM), const),
            pl.BlockSpec((1, HID), const),
            pl.BlockSpec((1, HID), const),
            pl.BlockSpec((1, HID), const),
            pl.BlockSpec((NP, HID + MEM), const),
            pl.BlockSpec((1, NP), const),
            pl.BlockSpec((SQ, MEM), const),
            pl.BlockSpec((SQ, MEM), const),
        ],
        out_specs=[
            pl.BlockSpec((BLOCK, HID), row),
            pl.BlockSpec((BLOCK, MEM), row),
        ],
        out_shape=[
            jax.ShapeDtypeStruct((B, HID), f32),
            jax.ShapeDtypeStruct((B, MEM), f32),
        ],
        compiler_params=pltpu.CompilerParams(
            dimension_semantics=("parallel",),
            vmem_limit_bytes=56 * 1024 * 1024,
        ),
    )(input, h_prev, m_prev,
      wh, b_h.reshape(1, HID), ln_g.reshape(1, HID), ln_b.reshape(1, HID),
      rows, b2, rrmat, ttmat)
    return h, m
